# baseline (speedup 1.0000x reference)
import functools

import jax
import jax.numpy as jnp
from jax import lax
from jax.experimental import pallas as pl
from jax.experimental.pallas import tpu as pltpu
from jax.experimental.pallas import tpu_sc as plsc

N = 10000
E = 160000
D_IN = 256
D_EMB = 512
H = D_IN // 2

NC = 2
NS = 16
L = 16

E_PAD = 161280
EPT = E_PAD // NS
CH = 40
NCH = EPT // CH
RS = 4
IB = 7
NBK = NCH // IB
N_ACC = 10040
PAD_ROW = N
WT = 10


def _edge_mlp_body(ea_ref, we1_ref, be1_ref, we2_ref, be2_ref, out_ref):
    ea = ea_ref[...].astype(jnp.bfloat16)
    h1 = jnp.dot(ea, we1_ref[...].astype(jnp.bfloat16),
                 preferred_element_type=jnp.float32)
    h1 = jnp.maximum(h1 + be1_ref[...], 0.0).astype(jnp.bfloat16)
    e = jnp.dot(h1, we2_ref[...].astype(jnp.bfloat16),
                preferred_element_type=jnp.float32)
    e = e + be2_ref[...]
    out_ref[0] = e[:, :H]
    out_ref[1] = e[:, H:]


def _edge_mlp(edge_attr, We1, be1, We2, be2, block_e=2016):
    grid = (E_PAD // block_e,)
    return pl.pallas_call(
        _edge_mlp_body,
        grid=grid,
        in_specs=[
            pl.BlockSpec((block_e, 4), lambda i: (i, 0)),
            pl.BlockSpec((4, D_IN), lambda i: (0, 0)),
            pl.BlockSpec((1, D_IN), lambda i: (0, 0)),
            pl.BlockSpec((D_IN, D_IN), lambda i: (0, 0)),
            pl.BlockSpec((1, D_IN), lambda i: (0, 0)),
        ],
        out_specs=pl.BlockSpec((NC, block_e, H), lambda i: (0, i, 0)),
        out_shape=jax.ShapeDtypeStruct((NC, E_PAD, H), jnp.float32),
    )(edge_attr, We1, be1.reshape(1, D_IN), We2, be2.reshape(1, D_IN))


def _xsplit_body(x_ref, o0_ref, o1_ref):
    o0_ref[...] = x_ref[:, :H]
    o1_ref[...] = x_ref[:, H:]


def _xsplit(x, block_n=2000):
    return pl.pallas_call(
        _xsplit_body,
        grid=(N // block_n,),
        in_specs=[pl.BlockSpec((block_n, D_IN), lambda i: (i, 0))],
        out_specs=[pl.BlockSpec((block_n, H), lambda i: (i, 0)),
                   pl.BlockSpec((block_n, H), lambda i: (i, 0))],
        out_shape=[jax.ShapeDtypeStruct((N, H), jnp.float32),
                   jax.ShapeDtypeStruct((N, H), jnp.float32)],
    )(x)


def _sc_body(x0_hbm, x1_hbm, e_hbm, src_hbm, dst_hbm, out_hbm,
             srcA, srcB, dstA, dstB,
             xb0, xb1, xb2, xb3, eb0, eb1, eb2, eb3,
             aggr_sh,
             sg0, sg1, sg2, sg3, se0, se1, se2, se3,
             ss0, ss1, ss2, ss3, semI):
    c = lax.axis_index("c")
    s = lax.axis_index("s")
    xbs = (xb0, xb1, xb2, xb3)
    ebs = (eb0, eb1, eb2, eb3)
    sgs = (sg0, sg1, sg2, sg3)
    ses = (se0, se1, se2, se3)
    sss = (ss0, ss1, ss2, ss3)
    zf = jnp.zeros((L,), jnp.float32)

    def _zrow(r, carry):
        for k in range(H // L):
            xb0[r, pl.ds(k * L, L)] = zf
        return carry

    lax.fori_loop(0, CH, _zrow, 0)
    for k in range(N_ACC // CH // NS + 1):
        chunk_id = s + k * NS

        @pl.when(chunk_id < N_ACC // CH)
        def _zero_chunk():
            pltpu.sync_copy(xb0, aggr_sh.at[pl.ds(chunk_id * CH, CH)])

    plsc.subcore_barrier()


    def _issue_gather(q, xb, semg):
        r = q % IB
        par = (q // IB) % 2

        @pl.when(jnp.logical_and(c == 0, par == 0))
        def _g00():
            pltpu.async_copy(x0_hbm.at[srcA.at[r]], xb, semg)

        @pl.when(jnp.logical_and(c == 0, par == 1))
        def _g01():
            pltpu.async_copy(x0_hbm.at[srcB.at[r]], xb, semg)

        @pl.when(jnp.logical_and(c == 1, par == 0))
        def _g10():
            pltpu.async_copy(x1_hbm.at[srcA.at[r]], xb, semg)

        @pl.when(jnp.logical_and(c == 1, par == 1))
        def _g11():
            pltpu.async_copy(x1_hbm.at[srcB.at[r]], xb, semg)

    def _issue_eload(q, eb, seme):
        pltpu.async_copy(e_hbm.at[c, pl.ds(s * EPT + q * CH, CH)], eb, seme)

    def _wait_in(xb, eb, semg, seme):
        pltpu.make_async_copy(x0_hbm.at[srcA.at[0]], xb, semg).wait()
        pltpu.make_async_copy(e_hbm.at[c, pl.ds(0, CH)], eb, seme).wait()

    def _issue_scatter(q, xb, sems):
        r = q % IB
        par = (q // IB) % 2

        @pl.when(par == 0)
        def _s0():
            pltpu.async_copy(xb, aggr_sh.at[dstA.at[r]], sems, add=True)

        @pl.when(par == 1)
        def _s1():
            pltpu.async_copy(xb, aggr_sh.at[dstB.at[r]], sems, add=True)

    def _wait_scatter(xb, sems):
        pltpu.make_async_copy(xb, aggr_sh.at[dstA.at[0]], sems).wait()

    def _wait_idx_block():
        pltpu.make_async_copy(src_hbm.at[s, 0], srcA, semI).wait()
        pltpu.make_async_copy(dst_hbm.at[s, 0], dstA, semI).wait()

    def _issue_idx_block(b):
        @pl.when(b % 2 == 0)
        def _ia():
            pltpu.async_copy(src_hbm.at[s, b], srcA, semI)
            pltpu.async_copy(dst_hbm.at[s, b], dstA, semI)

        @pl.when(b % 2 == 1)
        def _ib():
            pltpu.async_copy(src_hbm.at[s, b], srcB, semI)
            pltpu.async_copy(dst_hbm.at[s, b], dstB, semI)

    def _compute(xb, eb):
        def _rows(i, carry):
            for rr in range(2):
                for k in range(H // L):
                    sl = pl.ds(k * L, L)
                    xb[2 * i + rr, sl] = jnp.maximum(
                        xb[2 * i + rr, sl] + eb[2 * i + rr, sl], 0.0)
            return carry

        lax.fori_loop(0, CH // 2, _rows, 0)

    pltpu.sync_copy(src_hbm.at[s, 0], srcA)
    pltpu.sync_copy(dst_hbm.at[s, 0], dstA)
    _issue_idx_block(1)
    for t in range(RS - 1):
        _issue_gather(t, xbs[t], sgs[t])
        _issue_eload(t, ebs[t], ses[t])

    def _group(m, carry):
        for t in range(RS):
            j = RS * m + t
            q = j + (RS - 1)
            tq = (t + RS - 1) % RS
            _wait_in(xbs[t], ebs[t], sgs[t], ses[t])

            @pl.when(jnp.logical_and(q < NCH, q % IB == 0))
            def _wib():
                _wait_idx_block()

            @pl.when(jnp.logical_and(j % IB == 0, j // IB + 1 < NBK))
            def _pib():
                _issue_idx_block(j // IB + 1)

            @pl.when(jnp.logical_and(q < NCH, j >= 1))
            def _wsc():
                _wait_scatter(xbs[tq], sss[tq])

            @pl.when(q < NCH)
            def _iss():
                _issue_gather(q, xbs[tq], sgs[tq])
                _issue_eload(q, ebs[tq], ses[tq])

            _compute(xbs[t], ebs[t])
            _issue_scatter(j, xbs[t], sss[t])
        return carry

    lax.fori_loop(0, NCH // RS, _group, 0)

    for t in range(RS):
        _wait_scatter(xbs[t], sss[t])

    plsc.subcore_barrier()

    rows_per_wt = N // WT

    @pl.when(s < WT)
    def _write_phase():
        pltpu.sync_copy(aggr_sh.at[pl.ds(s * rows_per_wt, rows_per_wt)],
                        out_hbm.at[c, pl.ds(s * rows_per_wt, rows_per_wt)])


def _sc_gather_scatter(x0, x1, e2, src4, dst4):
    mesh = plsc.VectorSubcoreMesh(core_axis_name="c", subcore_axis_name="s",
                                  num_cores=NC, num_subcores=NS)
    fn = pl.kernel(
        _sc_body,
        out_type=jax.ShapeDtypeStruct((NC, N, H), jnp.float32),
        mesh=mesh,
        scratch_types=(
            [pltpu.VMEM((IB, CH), jnp.int32)] * 4
            + [pltpu.VMEM((CH, H), jnp.float32)] * 8
            + [pltpu.VMEM_SHARED((N_ACC, H), jnp.float32)]
            + [pltpu.SemaphoreType.DMA] * 13
        ),
    )
    return fn(x0, x1, e2, src4, dst4)


def _node_mlp_body(x_ref, a_ref, w1_ref, b1_ref, w2_ref, b2_ref, eps_ref,
                   out_ref):
    scale = 1.0 + eps_ref[0, 0]
    aggr = jnp.concatenate([a_ref[0], a_ref[1]], axis=1)
    h = scale * x_ref[...] + aggr
    m = jnp.dot(h, w1_ref[...], preferred_element_type=jnp.float32)
    m = jnp.maximum(m + b1_ref[...], 0.0)
    o = jnp.dot(m, w2_ref[...], preferred_element_type=jnp.float32)
    out_ref[...] = o + b2_ref[...]


def _node_mlp(x, aggr2, W1, b1, W2, b2, eps, block_n=2000):
    grid = (N // block_n,)
    return pl.pallas_call(
        _node_mlp_body,
        grid=grid,
        in_specs=[
            pl.BlockSpec((block_n, D_IN), lambda i: (i, 0)),
            pl.BlockSpec((NC, block_n, H), lambda i: (0, i, 0)),
            pl.BlockSpec((D_IN, D_EMB), lambda i: (0, 0)),
            pl.BlockSpec((1, D_EMB), lambda i: (0, 0)),
            pl.BlockSpec((D_EMB, D_EMB), lambda i: (0, 0)),
            pl.BlockSpec((1, D_EMB), lambda i: (0, 0)),
            pl.BlockSpec(memory_space=pltpu.SMEM),
        ],
        out_specs=pl.BlockSpec((block_n, D_EMB), lambda i: (i, 0)),
        out_shape=jax.ShapeDtypeStruct((N, D_EMB), jnp.float32),
    )(x, aggr2, W1, b1.reshape(1, D_EMB), W2, b2.reshape(1, D_EMB),
      eps.reshape(1, 1))


def kernel(x, edge_index, edge_attr, We1, be1, We2, be2, W1, b1, W2, b2, eps):
    npad = E_PAD - E
    src = edge_index[0].astype(jnp.int32)
    dst = edge_index[1].astype(jnp.int32)
    src_p = jnp.concatenate([src, jnp.zeros((npad,), jnp.int32)])
    dst_p = jnp.concatenate([dst, jnp.full((npad,), PAD_ROW, jnp.int32)])
    src4 = src_p.reshape(NS, NBK, IB, CH)
    dst4 = dst_p.reshape(NS, NBK, IB, CH)
    ea_p = jnp.concatenate(
        [edge_attr, jnp.zeros((npad, 4), jnp.float32)], axis=0)

    x0, x1 = _xsplit(x)

    e2 = _edge_mlp(ea_p, We1, be1, We2, be2)
    aggr2 = _sc_gather_scatter(x0, x1, e2, src4, dst4)
    return _node_mlp(x, aggr2, W1, b1, W2, b2, eps)


# speedup vs baseline: 1.2401x; 1.2401x over previous
import functools

import jax
import jax.numpy as jnp
from jax import lax
from jax.experimental import pallas as pl
from jax.experimental.pallas import tpu as pltpu
from jax.experimental.pallas import tpu_sc as plsc

N = 10000
E = 160000
D_IN = 256
D_EMB = 512
H = D_IN // 2

NC = 2
NS = 16
L = 16

E_PAD = 161280
EPT = E_PAD // NS
CH = 80
NCH = EPT // CH
RS = 2
IB = 7
NBK = NCH // IB
N_ACC = 10080
PAD_ROW = N
WT = 10


def _edge_mlp_body(ea_ref, we1_ref, be1_ref, we2_ref, be2_ref, out_ref):
    ea_t = ea_ref[...].astype(jnp.bfloat16)
    h1 = lax.dot_general(ea_t, we1_ref[...].astype(jnp.bfloat16),
                         (((0,), (0,)), ((), ())),
                         preferred_element_type=jnp.float32)
    h1 = jnp.maximum(h1 + be1_ref[...], 0.0).astype(jnp.bfloat16)
    e = jnp.dot(h1, we2_ref[...].astype(jnp.bfloat16),
                preferred_element_type=jnp.float32)
    e = e + be2_ref[...]
    out_ref[0] = e[:, :H]
    out_ref[1] = e[:, H:]


def _edge_mlp(edge_attr, We1, be1, We2, be2, block_e=2560):
    grid = (E_PAD // block_e,)
    return pl.pallas_call(
        _edge_mlp_body,
        grid=grid,
        in_specs=[
            pl.BlockSpec((4, block_e), lambda i: (0, i)),
            pl.BlockSpec((4, D_IN), lambda i: (0, 0)),
            pl.BlockSpec((1, D_IN), lambda i: (0, 0)),
            pl.BlockSpec((D_IN, D_IN), lambda i: (0, 0)),
            pl.BlockSpec((1, D_IN), lambda i: (0, 0)),
        ],
        out_specs=pl.BlockSpec((NC, block_e, H), lambda i: (0, i, 0)),
        out_shape=jax.ShapeDtypeStruct((NC, E_PAD, H), jnp.float32),
    )(edge_attr, We1, be1.reshape(1, D_IN), We2, be2.reshape(1, D_IN))


def _xsplit_body(x_ref, o0_ref, o1_ref):
    o0_ref[...] = x_ref[:, :H]
    o1_ref[...] = x_ref[:, H:]


def _xsplit(x, block_n=2000):
    return pl.pallas_call(
        _xsplit_body,
        grid=(N // block_n,),
        in_specs=[pl.BlockSpec((block_n, D_IN), lambda i: (i, 0))],
        out_specs=[pl.BlockSpec((block_n, H), lambda i: (i, 0)),
                   pl.BlockSpec((block_n, H), lambda i: (i, 0))],
        out_shape=[jax.ShapeDtypeStruct((N, H), jnp.float32),
                   jax.ShapeDtypeStruct((N, H), jnp.float32)],
    )(x)


def _sc_body(x0_hbm, x1_hbm, e_hbm, src_hbm, dst_hbm, out_hbm,
             srcA, srcB, dstA, dstB,
             xb0, xb1, eb0, eb1,
             aggr_sh,
             sg0, sg1, se0, se1,
             ss0, ss1, semI):
    c = lax.axis_index("c")
    s = lax.axis_index("s")
    xbs = (xb0, xb1)
    ebs = (eb0, eb1)
    sgs = (sg0, sg1)
    ses = (se0, se1)
    sss = (ss0, ss1)
    zf = jnp.zeros((L,), jnp.float32)

    def _zrow(r, carry):
        for k in range(H // L):
            xb0[r, pl.ds(k * L, L)] = zf
        return carry

    lax.fori_loop(0, CH, _zrow, 0)
    for k in range(N_ACC // CH // NS + 1):
        chunk_id = s + k * NS

        @pl.when(chunk_id < N_ACC // CH)
        def _zero_chunk():
            pltpu.sync_copy(xb0, aggr_sh.at[pl.ds(chunk_id * CH, CH)])

    plsc.subcore_barrier()


    def _issue_gather(q, xb, semg):
        r = q % IB
        par = (q // IB) % 2

        @pl.when(jnp.logical_and(c == 0, par == 0))
        def _g00():
            pltpu.async_copy(x0_hbm.at[srcA.at[r]], xb, semg)

        @pl.when(jnp.logical_and(c == 0, par == 1))
        def _g01():
            pltpu.async_copy(x0_hbm.at[srcB.at[r]], xb, semg)

        @pl.when(jnp.logical_and(c == 1, par == 0))
        def _g10():
            pltpu.async_copy(x1_hbm.at[srcA.at[r]], xb, semg)

        @pl.when(jnp.logical_and(c == 1, par == 1))
        def _g11():
            pltpu.async_copy(x1_hbm.at[srcB.at[r]], xb, semg)

    def _issue_eload(q, eb, seme):
        pltpu.async_copy(e_hbm.at[c, pl.ds(s * EPT + q * CH, CH)], eb, seme)

    def _wait_in(xb, eb, semg, seme):
        pltpu.make_async_copy(x0_hbm.at[srcA.at[0]], xb, semg).wait()
        pltpu.make_async_copy(e_hbm.at[c, pl.ds(0, CH)], eb, seme).wait()

    def _issue_scatter(q, xb, sems):
        r = q % IB
        par = (q // IB) % 2

        @pl.when(par == 0)
        def _s0():
            pltpu.async_copy(xb, aggr_sh.at[dstA.at[r]], sems, add=True)

        @pl.when(par == 1)
        def _s1():
            pltpu.async_copy(xb, aggr_sh.at[dstB.at[r]], sems, add=True)

    def _wait_scatter(xb, sems):
        pltpu.make_async_copy(xb, aggr_sh.at[dstA.at[0]], sems).wait()

    def _wait_idx_block():
        pltpu.make_async_copy(src_hbm.at[s, 0], srcA, semI).wait()
        pltpu.make_async_copy(dst_hbm.at[s, 0], dstA, semI).wait()

    def _issue_idx_block(b):
        @pl.when(b % 2 == 0)
        def _ia():
            pltpu.async_copy(src_hbm.at[s, b], srcA, semI)
            pltpu.async_copy(dst_hbm.at[s, b], dstA, semI)

        @pl.when(b % 2 == 1)
        def _ib():
            pltpu.async_copy(src_hbm.at[s, b], srcB, semI)
            pltpu.async_copy(dst_hbm.at[s, b], dstB, semI)

    def _compute(xb, eb):
        def _rows(i, carry):
            for rr in range(2):
                for k in range(H // L):
                    sl = pl.ds(k * L, L)
                    xb[2 * i + rr, sl] = jnp.maximum(
                        xb[2 * i + rr, sl] + eb[2 * i + rr, sl], 0.0)
            return carry

        lax.fori_loop(0, CH // 2, _rows, 0)

    pltpu.sync_copy(src_hbm.at[s, 0], srcA)
    pltpu.sync_copy(dst_hbm.at[s, 0], dstA)
    _issue_idx_block(1)
    for t in range(RS - 1):
        _issue_gather(t, xbs[t], sgs[t])
        _issue_eload(t, ebs[t], ses[t])

    def _group(m, carry):
        for t in range(RS):
            j = RS * m + t
            q = j + (RS - 1)
            tq = (t + RS - 1) % RS
            _wait_in(xbs[t], ebs[t], sgs[t], ses[t])

            @pl.when(jnp.logical_and(q < NCH, q % IB == 0))
            def _wib():
                _wait_idx_block()

            @pl.when(jnp.logical_and(j % IB == 0, j // IB + 1 < NBK))
            def _pib():
                _issue_idx_block(j // IB + 1)

            @pl.when(jnp.logical_and(q < NCH, j >= 1))
            def _wsc():
                _wait_scatter(xbs[tq], sss[tq])

            @pl.when(q < NCH)
            def _iss():
                _issue_gather(q, xbs[tq], sgs[tq])
                _issue_eload(q, ebs[tq], ses[tq])

            _compute(xbs[t], ebs[t])
            _issue_scatter(j, xbs[t], sss[t])
        return carry

    lax.fori_loop(0, NCH // RS, _group, 0)

    for t in range(RS):
        _wait_scatter(xbs[t], sss[t])

    plsc.subcore_barrier()

    rows_per_wt = N // WT

    @pl.when(s < WT)
    def _write_phase():
        pltpu.sync_copy(aggr_sh.at[pl.ds(s * rows_per_wt, rows_per_wt)],
                        out_hbm.at[c, pl.ds(s * rows_per_wt, rows_per_wt)])


def _sc_gather_scatter(x0, x1, e2, src4, dst4):
    mesh = plsc.VectorSubcoreMesh(core_axis_name="c", subcore_axis_name="s",
                                  num_cores=NC, num_subcores=NS)
    fn = pl.kernel(
        _sc_body,
        out_type=jax.ShapeDtypeStruct((NC, N, H), jnp.float32),
        mesh=mesh,
        scratch_types=(
            [pltpu.VMEM((IB, CH), jnp.int32)] * 4
            + [pltpu.VMEM((CH, H), jnp.float32)] * (2 * RS)
            + [pltpu.VMEM_SHARED((N_ACC, H), jnp.float32)]
            + [pltpu.SemaphoreType.DMA] * (3 * RS + 1)
        ),
    )
    return fn(x0, x1, e2, src4, dst4)


def _node_mlp_body(x_ref, a_ref, w1_ref, b1_ref, w2_ref, b2_ref, eps_ref,
                   out_ref):
    scale = 1.0 + eps_ref[0, 0]
    aggr = jnp.concatenate([a_ref[0], a_ref[1]], axis=1)
    h = scale * x_ref[...] + aggr
    m = jnp.dot(h, w1_ref[...], preferred_element_type=jnp.float32)
    m = jnp.maximum(m + b1_ref[...], 0.0)
    o = jnp.dot(m, w2_ref[...], preferred_element_type=jnp.float32)
    out_ref[...] = o + b2_ref[...]


def _node_mlp(x, aggr2, W1, b1, W2, b2, eps, block_n=2000):
    grid = (N // block_n,)
    return pl.pallas_call(
        _node_mlp_body,
        grid=grid,
        in_specs=[
            pl.BlockSpec((block_n, D_IN), lambda i: (i, 0)),
            pl.BlockSpec((NC, block_n, H), lambda i: (0, i, 0)),
            pl.BlockSpec((D_IN, D_EMB), lambda i: (0, 0)),
            pl.BlockSpec((1, D_EMB), lambda i: (0, 0)),
            pl.BlockSpec((D_EMB, D_EMB), lambda i: (0, 0)),
            pl.BlockSpec((1, D_EMB), lambda i: (0, 0)),
            pl.BlockSpec(memory_space=pltpu.SMEM),
        ],
        out_specs=pl.BlockSpec((block_n, D_EMB), lambda i: (i, 0)),
        out_shape=jax.ShapeDtypeStruct((N, D_EMB), jnp.float32),
    )(x, aggr2, W1, b1.reshape(1, D_EMB), W2, b2.reshape(1, D_EMB),
      eps.reshape(1, 1))


def kernel(x, edge_index, edge_attr, We1, be1, We2, be2, W1, b1, W2, b2, eps):
    npad = E_PAD - E
    src = edge_index[0].astype(jnp.int32)
    dst = edge_index[1].astype(jnp.int32)
    src_p = jnp.concatenate([src, jnp.zeros((npad,), jnp.int32)])
    dst_p = jnp.concatenate([dst, jnp.full((npad,), PAD_ROW, jnp.int32)])
    src4 = src_p.reshape(NS, NBK, IB, CH)
    dst4 = dst_p.reshape(NS, NBK, IB, CH)
    ea_t = jnp.concatenate(
        [edge_attr, jnp.zeros((npad, 4), jnp.float32)], axis=0).T

    x0, x1 = _xsplit(x)

    e2 = _edge_mlp(ea_t, We1, be1, We2, be2)
    aggr2 = _sc_gather_scatter(x0, x1, e2, src4, dst4)
    return _node_mlp(x, aggr2, W1, b1, W2, b2, eps)


# speedup vs baseline: 1.4584x; 1.1760x over previous
import functools

import jax
import jax.numpy as jnp
from jax import lax
from jax.experimental import pallas as pl
from jax.experimental.pallas import tpu as pltpu
from jax.experimental.pallas import tpu_sc as plsc

N = 10000
E = 160000
D_IN = 256
D_EMB = 512
H = D_IN // 2

NC = 2
NS = 16
L = 16

EPT = E // NS
CH = 80
NCH = EPT // CH
IB = 25
NB = NCH // IB
WT = 10


def _edge_mlp_body(ea_ref, we1_ref, be1_ref, we2_ref, be2_ref, out_ref):
    ea_t = ea_ref[...].astype(jnp.bfloat16)
    h1 = lax.dot_general(ea_t, we1_ref[...].astype(jnp.bfloat16),
                         (((0,), (0,)), ((), ())),
                         preferred_element_type=jnp.float32)
    h1 = jnp.maximum(h1 + be1_ref[...], 0.0).astype(jnp.bfloat16)
    e = jnp.dot(h1, we2_ref[...].astype(jnp.bfloat16),
                preferred_element_type=jnp.float32)
    e = e + be2_ref[...]
    out_ref[0] = e[:, :H]
    out_ref[1] = e[:, H:]


def _edge_mlp(edge_attr, We1, be1, We2, be2, block_e=3200):
    grid = (E // block_e,)
    return pl.pallas_call(
        _edge_mlp_body,
        grid=grid,
        in_specs=[
            pl.BlockSpec((4, block_e), lambda i: (0, i)),
            pl.BlockSpec((4, D_IN), lambda i: (0, 0)),
            pl.BlockSpec((1, D_IN), lambda i: (0, 0)),
            pl.BlockSpec((D_IN, D_IN), lambda i: (0, 0)),
            pl.BlockSpec((1, D_IN), lambda i: (0, 0)),
        ],
        out_specs=pl.BlockSpec((NC, block_e, H), lambda i: (0, i, 0)),
        out_shape=jax.ShapeDtypeStruct((NC, E, H), jnp.float32),
    )(edge_attr, We1, be1.reshape(1, D_IN), We2, be2.reshape(1, D_IN))


def _xsplit_body(x_ref, o0_ref, o1_ref):
    o0_ref[...] = x_ref[:, :H]
    o1_ref[...] = x_ref[:, H:]


def _xsplit(x, block_n=2000):
    return pl.pallas_call(
        _xsplit_body,
        grid=(N // block_n,),
        in_specs=[pl.BlockSpec((block_n, D_IN), lambda i: (i, 0))],
        out_specs=[pl.BlockSpec((block_n, H), lambda i: (i, 0)),
                   pl.BlockSpec((block_n, H), lambda i: (i, 0))],
        out_shape=[jax.ShapeDtypeStruct((N, H), jnp.float32),
                   jax.ShapeDtypeStruct((N, H), jnp.float32)],
    )(x)


def _sc_body(x0_hbm, x1_hbm, e_hbm, src_hbm, dst_hbm, out_hbm,
             src_v, dst_v, xb0, xb1, eb0, eb1, dcur0, dcur1,
             aggr_sh, semg0, seme0, semg1, seme1, sems0, sems1):
    c = lax.axis_index("c")
    s = lax.axis_index("s")
    zf = jnp.zeros((L,), jnp.float32)

    def _zrow(r, carry):
        for k in range(H // L):
            xb0[r, pl.ds(k * L, L)] = zf
        return carry

    lax.fori_loop(0, CH, _zrow, 0)
    for k in range(8):
        chunk_id = s + k * NS

        @pl.when(chunk_id < NCH)
        def _zero_chunk():
            pltpu.sync_copy(xb0, aggr_sh.at[pl.ds(chunk_id * CH, CH)])

    plsc.subcore_barrier()


    def _load_block(b):
        pltpu.sync_copy(src_hbm.at[s, b], src_v)
        pltpu.sync_copy(dst_hbm.at[s, b], dst_v)

    def _issue(q, xb, eb, semg, seme):
        r = q % IB
        idx = src_v.at[r]

        @pl.when(c == 0)
        def _g0():
            pltpu.async_copy(x0_hbm.at[idx], xb, semg)

        @pl.when(c == 1)
        def _g1():
            pltpu.async_copy(x1_hbm.at[idx], xb, semg)

        pltpu.async_copy(e_hbm.at[c, pl.ds(s * EPT + q * CH, CH)], eb, seme)

    def _wait(xb, eb, semg, seme):
        @pl.when(c == 0)
        def _w0():
            pltpu.make_async_copy(x0_hbm.at[src_v.at[0]], xb, semg).wait()

        @pl.when(c == 1)
        def _w1():
            pltpu.make_async_copy(x1_hbm.at[src_v.at[0]], xb, semg).wait()

        pltpu.make_async_copy(e_hbm.at[c, pl.ds(0, CH)], eb, seme).wait()

    def _snap_dst(q, dcur):
        r = q % IB
        for k in range(CH // L):
            sl = pl.ds(k * L, L)
            dcur[sl] = dst_v[r, sl]

    def _compute(xb, eb):
        def _rows(i, carry):
            for rr in range(2):
                for k in range(H // L):
                    sl = pl.ds(k * L, L)
                    xb[2 * i + rr, sl] = jnp.maximum(
                        xb[2 * i + rr, sl] + eb[2 * i + rr, sl], 0.0)
            return carry

        lax.fori_loop(0, CH // 2, _rows, 0)

    def _scatter_start(xb, dcur, sems):
        pltpu.async_copy(xb, aggr_sh.at[dcur], sems, add=True)

    def _scatter_wait(xb, dcur, sems):
        pltpu.make_async_copy(xb, aggr_sh.at[dcur], sems).wait()

    def _maybe_block(q):
        @pl.when(q % IB == 0)
        def _lb():
            _load_block(q // IB)

    _load_block(0)
    _issue(0, xb0, eb0, semg0, seme0)

    def _pair(m, carry):
        q0 = 2 * m
        q1 = q0 + 1
        q2 = q0 + 2
        _wait(xb0, eb0, semg0, seme0)
        _snap_dst(q0, dcur0)
        _maybe_block(q1)

        @pl.when(m > 0)
        def _ws1():
            _scatter_wait(xb1, dcur1, sems1)

        _issue(q1, xb1, eb1, semg1, seme1)
        _compute(xb0, eb0)
        _scatter_start(xb0, dcur0, sems0)
        _wait(xb1, eb1, semg1, seme1)
        _snap_dst(q1, dcur1)
        _maybe_block(q2)
        _scatter_wait(xb0, dcur0, sems0)
        _issue(q2, xb0, eb0, semg0, seme0)
        _compute(xb1, eb1)
        _scatter_start(xb1, dcur1, sems1)
        return carry

    lax.fori_loop(0, (NCH - 1) // 2, _pair, 0)

    _wait(xb0, eb0, semg0, seme0)
    _snap_dst(NCH - 1, dcur0)
    _compute(xb0, eb0)
    _scatter_wait(xb1, dcur1, sems1)
    _scatter_start(xb0, dcur0, sems0)
    _scatter_wait(xb0, dcur0, sems0)

    plsc.subcore_barrier()

    rows_per_wt = N // WT

    @pl.when(s < WT)
    def _write_phase():
        pltpu.sync_copy(aggr_sh.at[pl.ds(s * rows_per_wt, rows_per_wt)],
                        out_hbm.at[c, pl.ds(s * rows_per_wt, rows_per_wt)])


def _sc_gather_scatter(x0, x1, e2, src4, dst4):
    mesh = plsc.VectorSubcoreMesh(core_axis_name="c", subcore_axis_name="s",
                                  num_cores=NC, num_subcores=NS)
    fn = pl.kernel(
        _sc_body,
        out_type=jax.ShapeDtypeStruct((NC, N, H), jnp.float32),
        mesh=mesh,
        scratch_types=[
            pltpu.VMEM((IB, CH), jnp.int32),
            pltpu.VMEM((IB, CH), jnp.int32),
            pltpu.VMEM((CH, H), jnp.float32),
            pltpu.VMEM((CH, H), jnp.float32),
            pltpu.VMEM((CH, H), jnp.float32),
            pltpu.VMEM((CH, H), jnp.float32),
            pltpu.VMEM((CH,), jnp.int32),
            pltpu.VMEM((CH,), jnp.int32),
            pltpu.VMEM_SHARED((N, H), jnp.float32),
            pltpu.SemaphoreType.DMA,
            pltpu.SemaphoreType.DMA,
            pltpu.SemaphoreType.DMA,
            pltpu.SemaphoreType.DMA,
            pltpu.SemaphoreType.DMA,
            pltpu.SemaphoreType.DMA,
        ],
    )
    return fn(x0, x1, e2, src4, dst4)


def _node_mlp_body(x_ref, a_ref, w1_ref, b1_ref, w2_ref, b2_ref, eps_ref,
                   out_ref):
    scale = 1.0 + eps_ref[0, 0]
    aggr = jnp.concatenate([a_ref[0], a_ref[1]], axis=1)
    h = scale * x_ref[...] + aggr
    m = jnp.dot(h, w1_ref[...], preferred_element_type=jnp.float32)
    m = jnp.maximum(m + b1_ref[...], 0.0)
    o = jnp.dot(m, w2_ref[...], preferred_element_type=jnp.float32)
    out_ref[...] = o + b2_ref[...]


def _node_mlp(x, aggr2, W1, b1, W2, b2, eps, block_n=2000):
    grid = (N // block_n,)
    return pl.pallas_call(
        _node_mlp_body,
        grid=grid,
        in_specs=[
            pl.BlockSpec((block_n, D_IN), lambda i: (i, 0)),
            pl.BlockSpec((NC, block_n, H), lambda i: (0, i, 0)),
            pl.BlockSpec((D_IN, D_EMB), lambda i: (0, 0)),
            pl.BlockSpec((1, D_EMB), lambda i: (0, 0)),
            pl.BlockSpec((D_EMB, D_EMB), lambda i: (0, 0)),
            pl.BlockSpec((1, D_EMB), lambda i: (0, 0)),
            pl.BlockSpec(memory_space=pltpu.SMEM),
        ],
        out_specs=pl.BlockSpec((block_n, D_EMB), lambda i: (i, 0)),
        out_shape=jax.ShapeDtypeStruct((N, D_EMB), jnp.float32),
    )(x, aggr2, W1, b1.reshape(1, D_EMB), W2, b2.reshape(1, D_EMB),
      eps.reshape(1, 1))


def kernel(x, edge_index, edge_attr, We1, be1, We2, be2, W1, b1, W2, b2, eps):
    src = edge_index[0].astype(jnp.int32)
    dst = edge_index[1].astype(jnp.int32)
    src4 = src.reshape(NS, NB, IB, CH)
    dst4 = dst.reshape(NS, NB, IB, CH)
    ea_t = edge_attr.T
    x0, x1 = _xsplit(x)

    e2 = _edge_mlp(ea_t, We1, be1, We2, be2)
    aggr2 = _sc_gather_scatter(x0, x1, e2, src4, dst4)
    return _node_mlp(x, aggr2, W1, b1, W2, b2, eps)
